# baseline (speedup 1.0000x reference)
import functools

import jax
import jax.numpy as jnp
from jax import lax
from jax.experimental import pallas as pl
from jax.experimental.pallas import tpu as pltpu


def _fps_body(px_ref, py_ref, pz_ref, out_ref, *, npoint, n):
    b = px_ref.shape[0]
    px = px_ref[...]
    py = py_ref[...]
    pz = pz_ref[...]
    flat = lax.broadcasted_iota(jnp.int32, (b, n), 1)
    ninf = jnp.float32(-jnp.inf)

    def body(i, carry):
        dists, far = carry
        sel = flat == far
        cx = jnp.max(jnp.where(sel, px, ninf), axis=1, keepdims=True)
        cy = jnp.max(jnp.where(sel, py, ninf), axis=1, keepdims=True)
        cz = jnp.max(jnp.where(sel, pz, ninf), axis=1, keepdims=True)
        c = jnp.concatenate([cx, cy, cz], axis=1)
        out_ref[pl.ds(i, 1)] = c[None]
        dx = px - cx
        dy = py - cy
        dz = pz - cz
        d = (dx * dx + dy * dy) + dz * dz
        dists = jnp.minimum(dists, d)
        mx = jnp.max(dists, axis=1, keepdims=True)
        far = jnp.min(jnp.where(dists == mx, flat, n), axis=1, keepdims=True)
        return dists, far

    dists0 = jnp.full((b, n), 1e10, dtype=jnp.float32)
    far0 = jnp.zeros((b, 1), dtype=jnp.int32)
    lax.fori_loop(0, npoint, body, (dists0, far0))


def _fps_call(px, py, pz, npoint):
    b, n = px.shape
    fn = pl.pallas_call(
        functools.partial(_fps_body, npoint=npoint, n=n),
        out_shape=jax.ShapeDtypeStruct((npoint, b, 3), jnp.float32),
    )
    return jnp.transpose(fn(px, py, pz), (1, 0, 2))


def _conv_body(centers_ref, post_ref, table_ref, *refs, r2, n, blk, ctab):
    out_ref = refs[-1]
    w_refs = refs[:-1]
    cx = centers_ref[0, :, 0:1]
    cy = centers_ref[0, :, 1:2]
    cz = centers_ref[0, :, 2:3]
    px = post_ref[0, 0:1, :]
    py = post_ref[0, 1:2, :]
    pz = post_ref[0, 2:3, :]
    dx = cx - px
    dy = cy - py
    dz = cz - pz
    d2 = (dx * dx + dy * dy) + dz * dz
    iota = lax.broadcasted_iota(jnp.int32, (blk, n), 1)
    midx = jnp.where(d2 > r2, n, iota)
    idx1 = jnp.min(midx, axis=1, keepdims=True)
    idx2 = jnp.min(jnp.where(midx > idx1, midx, n), axis=1, keepdims=True)
    idx2 = jnp.where(idx2 == n, idx1, idx2)
    table = table_ref[0]
    oh1 = (iota == idx1).astype(jnp.float32)
    oh2 = (iota == idx2).astype(jnp.float32)
    row1 = jnp.dot(oh1, table, preferred_element_type=jnp.float32,
                   precision=lax.Precision.HIGHEST)
    row2 = jnp.dot(oh2, table, preferred_element_type=jnp.float32,
                   precision=lax.Precision.HIGHEST)
    cpad = jnp.concatenate(
        [centers_ref[0], jnp.zeros((blk, ctab - 3), jnp.float32)], axis=1)
    g = jnp.concatenate([row1 - cpad, row2 - cpad], axis=0)
    for i in range(0, len(w_refs), 2):
        w = w_refs[i][...]
        bias = w_refs[i + 1][...]
        g = jnp.maximum(jnp.dot(g, w, preferred_element_type=jnp.float32) + bias, 0.0)
    out_ref[0] = jnp.maximum(g[:blk], g[blk:])


def _conv_call(centers, post, table, layers, radius, blk):
    b, npoint, _ = centers.shape
    n = post.shape[2]
    ctab = table.shape[2]
    cout = layers[-1][0].shape[1]
    w_args = []
    w_specs = []
    for (w, bias) in layers:
        w_args.append(w)
        w_specs.append(pl.BlockSpec(w.shape, lambda bb, j: (0, 0)))
        bias2 = bias.reshape(1, -1)
        w_args.append(bias2)
        w_specs.append(pl.BlockSpec(bias2.shape, lambda bb, j: (0, 0)))
    fn = pl.pallas_call(
        functools.partial(_conv_body, r2=radius * radius, n=n, blk=blk, ctab=ctab),
        grid=(b, npoint // blk),
        in_specs=[
            pl.BlockSpec((1, blk, 3), lambda bb, j: (bb, j, 0)),
            pl.BlockSpec((1, 3, n), lambda bb, j: (bb, 0, 0)),
            pl.BlockSpec((1, n, ctab), lambda bb, j: (bb, 0, 0)),
            *w_specs,
        ],
        out_specs=pl.BlockSpec((1, blk, cout), lambda bb, j: (bb, j, 0)),
        out_shape=jax.ShapeDtypeStruct((b, npoint, cout), jnp.float32),
    )
    return fn(centers, post, table, *w_args)


def kernel(x, features, params):
    pos = x[:, :, :3]
    feat = jnp.transpose(features, (0, 2, 1))

    px, py, pz = pos[:, :, 0], pos[:, :, 1], pos[:, :, 2]
    pos2 = _fps_call(px, py, pz, npoint=2048)
    post1 = jnp.transpose(pos, (0, 2, 1))
    table1 = jnp.concatenate([pos, features], axis=-1)
    nf1 = _conv_call(pos2, post1, table1, params["l1"], radius=0.5, blk=128)
    feat2 = jnp.transpose(nf1, (0, 2, 1))

    pos3 = _fps_call(pos2[:, :, 0], pos2[:, :, 1], pos2[:, :, 2], npoint=512)
    post2 = jnp.transpose(pos2, (0, 2, 1))
    table2 = jnp.concatenate([pos2, nf1], axis=-1)
    nf2 = _conv_call(pos3, post2, table2, params["l2"], radius=1.0, blk=128)
    feat3 = jnp.transpose(nf2, (0, 2, 1))

    return (pos, feat, pos2, feat2, pos3, feat3)


# speedup vs baseline: 1.0769x; 1.0769x over previous
import functools

import jax
import jax.numpy as jnp
from jax import lax
from jax.experimental import pallas as pl
from jax.experimental.pallas import tpu as pltpu


def _rmax(v):
    return jnp.max(jnp.max(v, axis=1, keepdims=True), axis=0, keepdims=True)


def _rmin(v):
    return jnp.min(jnp.min(v, axis=1, keepdims=True), axis=0, keepdims=True)


def _fps_body(px_ref, py_ref, pz_ref, out_ref, *, npoint, n):
    b = px_ref.shape[0]
    m = n // 8
    px = [px_ref[i] for i in range(b)]
    py = [py_ref[i] for i in range(b)]
    pz = [pz_ref[i] for i in range(b)]
    flat = (lax.broadcasted_iota(jnp.int32, (8, m), 0) * m
            + lax.broadcasted_iota(jnp.int32, (8, m), 1))
    ninf = jnp.float32(-jnp.inf)

    def body(i, carry):
        dists, far = carry
        new_dists, new_far = [], []
        for k in range(b):
            sel = flat == far[k]
            cx = _rmax(jnp.where(sel, px[k], ninf))
            cy = _rmax(jnp.where(sel, py[k], ninf))
            cz = _rmax(jnp.where(sel, pz[k], ninf))
            c = jnp.concatenate([cx, cy, cz], axis=1)
            out_ref[pl.ds(i, 1), k] = c
            dx = px[k] - cx
            dy = py[k] - cy
            dz = pz[k] - cz
            d = (dx * dx + dy * dy) + dz * dz
            dk = jnp.minimum(dists[k], d)
            mx = _rmax(dk)
            new_far.append(_rmin(jnp.where(dk == mx, flat, n)))
            new_dists.append(dk)
        return new_dists, new_far

    dists0 = [jnp.full((8, m), 1e10, dtype=jnp.float32)] * b
    far0 = [jnp.zeros((1, 1), dtype=jnp.int32)] * b
    lax.fori_loop(0, npoint, body, (dists0, far0))


def _fps_call(px, py, pz, npoint):
    b, n = px.shape
    fn = pl.pallas_call(
        functools.partial(_fps_body, npoint=npoint, n=n),
        out_shape=jax.ShapeDtypeStruct((npoint, b, 3), jnp.float32),
    )
    shape = (b, 8, n // 8)
    out = fn(px.reshape(shape), py.reshape(shape), pz.reshape(shape))
    return jnp.transpose(out, (1, 0, 2))


def _conv_body(centers_ref, post_ref, table_ref, *refs, r2, n, blk, ctab):
    out_ref = refs[-1]
    w_refs = refs[:-1]
    cx = centers_ref[0, :, 0:1]
    cy = centers_ref[0, :, 1:2]
    cz = centers_ref[0, :, 2:3]
    px = post_ref[0, 0:1, :]
    py = post_ref[0, 1:2, :]
    pz = post_ref[0, 2:3, :]
    dx = cx - px
    dy = cy - py
    dz = cz - pz
    d2 = (dx * dx + dy * dy) + dz * dz
    iota = lax.broadcasted_iota(jnp.int32, (blk, n), 1)
    midx = jnp.where(d2 > r2, n, iota)
    idx1 = jnp.min(midx, axis=1, keepdims=True)
    idx2 = jnp.min(jnp.where(midx > idx1, midx, n), axis=1, keepdims=True)
    idx2 = jnp.where(idx2 == n, idx1, idx2)
    table = table_ref[0]
    oh1 = (iota == idx1).astype(jnp.float32)
    oh2 = (iota == idx2).astype(jnp.float32)
    row1 = jnp.dot(oh1, table, preferred_element_type=jnp.float32,
                   precision=lax.Precision.HIGHEST)
    row2 = jnp.dot(oh2, table, preferred_element_type=jnp.float32,
                   precision=lax.Precision.HIGHEST)
    cpad = jnp.concatenate(
        [centers_ref[0], jnp.zeros((blk, ctab - 3), jnp.float32)], axis=1)
    g = jnp.concatenate([row1 - cpad, row2 - cpad], axis=0)
    for i in range(0, len(w_refs), 2):
        w = w_refs[i][...]
        bias = w_refs[i + 1][...]
        g = jnp.maximum(jnp.dot(g, w, preferred_element_type=jnp.float32) + bias, 0.0)
    out_ref[0] = jnp.maximum(g[:blk], g[blk:])


def _conv_call(centers, post, table, layers, radius, blk):
    b, npoint, _ = centers.shape
    n = post.shape[2]
    ctab = table.shape[2]
    cout = layers[-1][0].shape[1]
    w_args = []
    w_specs = []
    for (w, bias) in layers:
        w_args.append(w)
        w_specs.append(pl.BlockSpec(w.shape, lambda bb, j: (0, 0)))
        bias2 = bias.reshape(1, -1)
        w_args.append(bias2)
        w_specs.append(pl.BlockSpec(bias2.shape, lambda bb, j: (0, 0)))
    fn = pl.pallas_call(
        functools.partial(_conv_body, r2=radius * radius, n=n, blk=blk, ctab=ctab),
        grid=(b, npoint // blk),
        in_specs=[
            pl.BlockSpec((1, blk, 3), lambda bb, j: (bb, j, 0)),
            pl.BlockSpec((1, 3, n), lambda bb, j: (bb, 0, 0)),
            pl.BlockSpec((1, n, ctab), lambda bb, j: (bb, 0, 0)),
            *w_specs,
        ],
        out_specs=pl.BlockSpec((1, blk, cout), lambda bb, j: (bb, j, 0)),
        out_shape=jax.ShapeDtypeStruct((b, npoint, cout), jnp.float32),
    )
    return fn(centers, post, table, *w_args)


def kernel(x, features, params):
    pos = x[:, :, :3]
    feat = jnp.transpose(features, (0, 2, 1))

    px, py, pz = pos[:, :, 0], pos[:, :, 1], pos[:, :, 2]
    pos2 = _fps_call(px, py, pz, npoint=2048)
    post1 = jnp.transpose(pos, (0, 2, 1))
    table1 = jnp.concatenate([pos, features], axis=-1)
    nf1 = _conv_call(pos2, post1, table1, params["l1"], radius=0.5, blk=128)
    feat2 = jnp.transpose(nf1, (0, 2, 1))

    pos3 = _fps_call(pos2[:, :, 0], pos2[:, :, 1], pos2[:, :, 2], npoint=512)
    post2 = jnp.transpose(pos2, (0, 2, 1))
    table2 = jnp.concatenate([pos2, nf1], axis=-1)
    nf2 = _conv_call(pos3, post2, table2, params["l2"], radius=1.0, blk=128)
    feat3 = jnp.transpose(nf2, (0, 2, 1))

    return (pos, feat, pos2, feat2, pos3, feat3)


# speedup vs baseline: 1.2236x; 1.1362x over previous
import functools

import jax
import jax.numpy as jnp
from jax import lax
from jax.experimental import pallas as pl
from jax.experimental.pallas import tpu as pltpu
from jax.experimental.pallas import tpu_sc as plsc

_SC_NC = 2
_SC_NS = 16
_SC_NW = _SC_NC * _SC_NS
_SC_CHUNK = 128


def _rmax(v):
    return jnp.max(jnp.max(v, axis=1, keepdims=True), axis=0, keepdims=True)


def _rmin(v):
    return jnp.min(jnp.min(v, axis=1, keepdims=True), axis=0, keepdims=True)


def _fps_body(px_ref, py_ref, pz_ref, out_ref, *, npoint, n):
    b = px_ref.shape[0]
    m = n // 8
    px = [px_ref[i] for i in range(b)]
    py = [py_ref[i] for i in range(b)]
    pz = [pz_ref[i] for i in range(b)]
    flat = (lax.broadcasted_iota(jnp.int32, (8, m), 0) * m
            + lax.broadcasted_iota(jnp.int32, (8, m), 1))
    ninf = jnp.float32(-jnp.inf)

    def body(i, carry):
        dists, far = carry
        new_dists, new_far = [], []
        for k in range(b):
            sel = flat == far[k]
            cx = _rmax(jnp.where(sel, px[k], ninf))
            cy = _rmax(jnp.where(sel, py[k], ninf))
            cz = _rmax(jnp.where(sel, pz[k], ninf))
            c = jnp.concatenate([cx, cy, cz], axis=1)
            out_ref[pl.ds(i, 1), k] = c
            dx = px[k] - cx
            dy = py[k] - cy
            dz = pz[k] - cz
            d = (dx * dx + dy * dy) + dz * dz
            dk = jnp.minimum(dists[k], d)
            mx = _rmax(dk)
            new_far.append(_rmin(jnp.where(dk == mx, flat, n)))
            new_dists.append(dk)
        return new_dists, new_far

    dists0 = [jnp.full((8, m), 1e10, dtype=jnp.float32)] * b
    far0 = [jnp.zeros((1, 1), dtype=jnp.int32)] * b
    lax.fori_loop(0, npoint, body, (dists0, far0))


def _fps_call(px, py, pz, npoint):
    b, n = px.shape
    fn = pl.pallas_call(
        functools.partial(_fps_body, npoint=npoint, n=n),
        out_shape=jax.ShapeDtypeStruct((npoint, b, 3), jnp.float32),
    )
    shape = (b, 8, n // 8)
    out = fn(px.reshape(shape), py.reshape(shape), pz.reshape(shape))
    return jnp.transpose(out, (1, 0, 2))


def _bq_body(centers_ref, post_ref, out_ref, *, r2, n, blk):
    cx = centers_ref[0, :, 0:1]
    cy = centers_ref[0, :, 1:2]
    cz = centers_ref[0, :, 2:3]
    px = post_ref[0, 0:1, :]
    py = post_ref[0, 1:2, :]
    pz = post_ref[0, 2:3, :]
    dx = cx - px
    dy = cy - py
    dz = cz - pz
    d2 = (dx * dx + dy * dy) + dz * dz
    iota = lax.broadcasted_iota(jnp.int32, (blk, n), 1)
    midx = jnp.where(d2 > r2, n, iota)
    idx1 = jnp.min(midx, axis=1, keepdims=True)
    idx2 = jnp.min(jnp.where(midx > idx1, midx, n), axis=1, keepdims=True)
    idx2 = jnp.where(idx2 == n, idx1, idx2)
    base = pl.program_id(0) * n
    out_ref[0] = jnp.concatenate([idx1 + base, idx2 + base], axis=1)


def _bq_call(centers, post, radius, blk):
    b, npoint, _ = centers.shape
    n = post.shape[2]
    fn = pl.pallas_call(
        functools.partial(_bq_body, r2=radius * radius, n=n, blk=blk),
        grid=(b, npoint // blk),
        in_specs=[
            pl.BlockSpec((1, blk, 3), lambda bb, j: (bb, j, 0)),
            pl.BlockSpec((1, 3, n), lambda bb, j: (bb, 0, 0)),
        ],
        out_specs=pl.BlockSpec((1, blk, 2), lambda bb, j: (bb, j, 0)),
        out_shape=jax.ShapeDtypeStruct((b, npoint, 2), jnp.int32),
    )
    return fn(centers, post)


def _sc_gather(table, idx, dpad):
    m = idx.shape[0]
    mpw = m // _SC_NW
    ch = min(mpw, _SC_CHUNK)
    nch = mpw // ch
    idx2d = idx.reshape(m // ch, ch)
    mesh = plsc.VectorSubcoreMesh(
        core_axis_name="c", subcore_axis_name="s",
        num_cores=_SC_NC, num_subcores=_SC_NS)

    def body(table_hbm, idx_hbm, out_hbm, idx_v, rows_v, sem):
        wid = lax.axis_index("s") * _SC_NC + lax.axis_index("c")
        row0 = wid * nch
        pltpu.sync_copy(idx_hbm.at[pl.ds(row0, nch)], idx_v)
        copies = [
            pltpu.async_copy(table_hbm.at[idx_v.at[j]], rows_v.at[j], sem)
            for j in range(nch)
        ]
        for cpy in copies:
            cpy.wait()
        for j in range(nch):
            pltpu.sync_copy(rows_v.at[j], out_hbm.at[pl.ds((row0 + j) * ch, ch)])

    fn = pl.kernel(
        body,
        out_type=jax.ShapeDtypeStruct((m, dpad), jnp.float32),
        mesh=mesh,
        scratch_types=[
            pltpu.VMEM((nch, ch), jnp.int32),
            pltpu.VMEM((nch, ch, dpad), jnp.float32),
            pltpu.SemaphoreType.DMA,
        ],
    )
    return fn(table, idx2d)


def _mlp_body(rows1_ref, rows2_ref, centers_ref, *refs, ctab, blk):
    out_ref = refs[-1]
    w_refs = refs[:-1]
    r1 = rows1_ref[0][:, :ctab]
    r2 = rows2_ref[0][:, :ctab]
    cpad = jnp.concatenate(
        [centers_ref[0], jnp.zeros((blk, ctab - 3), jnp.float32)], axis=1)
    g = jnp.concatenate([r1 - cpad, r2 - cpad], axis=0)
    for i in range(0, len(w_refs), 2):
        w = w_refs[i][...]
        bias = w_refs[i + 1][...]
        g = jnp.maximum(jnp.dot(g, w, preferred_element_type=jnp.float32) + bias, 0.0)
    out_ref[0] = jnp.maximum(g[:blk], g[blk:])


def _mlp_call(rows1, rows2, centers, layers, blk, ctab):
    b, npoint, dpad = rows1.shape
    cout = layers[-1][0].shape[1]
    w_args = []
    w_specs = []
    for (w, bias) in layers:
        w_args.append(w)
        w_specs.append(pl.BlockSpec(w.shape, lambda bb, j: (0, 0)))
        bias2 = bias.reshape(1, -1)
        w_args.append(bias2)
        w_specs.append(pl.BlockSpec(bias2.shape, lambda bb, j: (0, 0)))
    fn = pl.pallas_call(
        functools.partial(_mlp_body, ctab=ctab, blk=blk),
        grid=(b, npoint // blk),
        in_specs=[
            pl.BlockSpec((1, blk, dpad), lambda bb, j: (bb, j, 0)),
            pl.BlockSpec((1, blk, dpad), lambda bb, j: (bb, j, 0)),
            pl.BlockSpec((1, blk, 3), lambda bb, j: (bb, j, 0)),
            *w_specs,
        ],
        out_specs=pl.BlockSpec((1, blk, cout), lambda bb, j: (bb, j, 0)),
        out_shape=jax.ShapeDtypeStruct((b, npoint, cout), jnp.float32),
    )
    return fn(rows1, rows2, centers, *w_args)


def _set_conv(centers, post, table, layers, radius, blk):
    b, npoint, _ = centers.shape
    n = post.shape[2]
    ctab = table.shape[2]
    dpad = 128
    idx = _bq_call(centers, post, radius, blk)
    table_pad = jnp.concatenate(
        [table, jnp.zeros((b, n, dpad - ctab), jnp.float32)], axis=-1)
    rows = _sc_gather(table_pad.reshape(b * n, dpad), idx.reshape(-1), dpad)
    rows = rows.reshape(b, npoint, 2, dpad)
    return _mlp_call(rows[:, :, 0], rows[:, :, 1], centers, layers, blk, ctab)


def kernel(x, features, params):
    pos = x[:, :, :3]
    feat = jnp.transpose(features, (0, 2, 1))

    px, py, pz = pos[:, :, 0], pos[:, :, 1], pos[:, :, 2]
    pos2 = _fps_call(px, py, pz, npoint=2048)
    post1 = jnp.transpose(pos, (0, 2, 1))
    table1 = jnp.concatenate([pos, features], axis=-1)
    nf1 = _set_conv(pos2, post1, table1, params["l1"], radius=0.5, blk=128)
    feat2 = jnp.transpose(nf1, (0, 2, 1))

    pos3 = _fps_call(pos2[:, :, 0], pos2[:, :, 1], pos2[:, :, 2], npoint=512)
    post2 = jnp.transpose(pos2, (0, 2, 1))
    table2 = jnp.concatenate([pos2, nf1], axis=-1)
    nf2 = _set_conv(pos3, post2, table2, params["l2"], radius=1.0, blk=128)
    feat3 = jnp.transpose(nf2, (0, 2, 1))

    return (pos, feat, pos2, feat2, pos3, feat3)


# speedup vs baseline: 1.2241x; 1.0004x over previous
import functools

import jax
import jax.numpy as jnp
from jax import lax
from jax.experimental import pallas as pl
from jax.experimental.pallas import tpu as pltpu
from jax.experimental.pallas import tpu_sc as plsc

_SC_NC = 2
_SC_NS = 16
_SC_NW = _SC_NC * _SC_NS
_SC_CHUNK = 128


def _rmax(v):
    return jnp.max(jnp.max(v, axis=1, keepdims=True), axis=0, keepdims=True)


def _rmin(v):
    return jnp.min(jnp.min(v, axis=1, keepdims=True), axis=0, keepdims=True)


def _fps_body(px_ref, py_ref, pz_ref, out_ref, *, npoint, n):
    b = px_ref.shape[0]
    m = n // 8
    px = [px_ref[i] for i in range(b)]
    py = [py_ref[i] for i in range(b)]
    pz = [pz_ref[i] for i in range(b)]
    flat = (lax.broadcasted_iota(jnp.int32, (8, m), 0) * m
            + lax.broadcasted_iota(jnp.int32, (8, m), 1))
    ninf = jnp.float32(-jnp.inf)

    def step(i, dists, fars):
        new_dists, new_fars = [], []
        for k in range(b):
            sel = flat == fars[k]
            cx = _rmax(jnp.where(sel, px[k], ninf))
            cy = _rmax(jnp.where(sel, py[k], ninf))
            cz = _rmax(jnp.where(sel, pz[k], ninf))
            c = jnp.concatenate([cx, cy, cz], axis=1)
            out_ref[pl.ds(i, 1), k] = c
            dx = px[k] - cx
            dy = py[k] - cy
            dz = pz[k] - cz
            d = (dx * dx + dy * dy) + dz * dz
            dk = jnp.minimum(dists[k], d)
            mx = _rmax(dk)
            new_fars.append(_rmin(jnp.where(dk == mx, flat, n)))
            new_dists.append(dk)
        return new_dists, new_fars

    def body(j, carry):
        dists, fars = carry
        dists, fars = step(2 * j, dists, fars)
        dists, fars = step(2 * j + 1, dists, fars)
        return dists, fars

    dists0 = [jnp.full((8, m), 1e10, dtype=jnp.float32)] * b
    far0 = [jnp.zeros((1, 1), dtype=jnp.int32)] * b
    lax.fori_loop(0, npoint // 2, body, (dists0, far0))


def _fps_call(px, py, pz, npoint):
    b, n = px.shape
    fn = pl.pallas_call(
        functools.partial(_fps_body, npoint=npoint, n=n),
        out_shape=jax.ShapeDtypeStruct((npoint, b, 3), jnp.float32),
    )
    shape = (b, 8, n // 8)
    return fn(px.reshape(shape), py.reshape(shape), pz.reshape(shape))


def _bq_body(centers_ref, post_ref, out_ref, *, r2, n, blk):
    cx = centers_ref[0, :, 0:1]
    cy = centers_ref[0, :, 1:2]
    cz = centers_ref[0, :, 2:3]
    px = post_ref[0, 0:1, :]
    py = post_ref[0, 1:2, :]
    pz = post_ref[0, 2:3, :]
    dx = cx - px
    dy = cy - py
    dz = cz - pz
    d2 = (dx * dx + dy * dy) + dz * dz
    iota = lax.broadcasted_iota(jnp.int32, (blk, n), 1)
    midx = jnp.where(d2 > r2, n, iota)
    idx1 = jnp.min(midx, axis=1, keepdims=True)
    idx2 = jnp.min(jnp.where(midx > idx1, midx, n), axis=1, keepdims=True)
    idx2 = jnp.where(idx2 == n, idx1, idx2)
    base = pl.program_id(0) * n
    out_ref[0] = jnp.concatenate([idx1 + base, idx2 + base], axis=1)


def _bq_call(centers, post, radius, blk):
    b, npoint, _ = centers.shape
    n = post.shape[2]
    fn = pl.pallas_call(
        functools.partial(_bq_body, r2=radius * radius, n=n, blk=blk),
        grid=(b, npoint // blk),
        in_specs=[
            pl.BlockSpec((1, blk, 3), lambda bb, j: (bb, j, 0)),
            pl.BlockSpec((1, 3, n), lambda bb, j: (bb, 0, 0)),
        ],
        out_specs=pl.BlockSpec((1, blk, 2), lambda bb, j: (bb, j, 0)),
        out_shape=jax.ShapeDtypeStruct((b, npoint, 2), jnp.int32),
    )
    return fn(centers, post)


def _sc_gather(table, idx, dpad):
    m = idx.shape[0]
    mpw = m // _SC_NW
    ch = min(mpw, _SC_CHUNK)
    nch = mpw // ch
    idx2d = idx.reshape(m // ch, ch)
    mesh = plsc.VectorSubcoreMesh(
        core_axis_name="c", subcore_axis_name="s",
        num_cores=_SC_NC, num_subcores=_SC_NS)

    def body(table_hbm, idx_hbm, out_hbm, idx_v, rows_v, sem):
        wid = lax.axis_index("s") * _SC_NC + lax.axis_index("c")
        row0 = wid * nch
        pltpu.sync_copy(idx_hbm.at[pl.ds(row0, nch)], idx_v)
        copies = [
            pltpu.async_copy(table_hbm.at[idx_v.at[j]], rows_v.at[j], sem)
            for j in range(nch)
        ]
        for cpy in copies:
            cpy.wait()
        for j in range(nch):
            pltpu.sync_copy(rows_v.at[j], out_hbm.at[pl.ds((row0 + j) * ch, ch)])

    fn = pl.kernel(
        body,
        out_type=jax.ShapeDtypeStruct((m, dpad), jnp.float32),
        mesh=mesh,
        scratch_types=[
            pltpu.VMEM((nch, ch), jnp.int32),
            pltpu.VMEM((nch, ch, dpad), jnp.float32),
            pltpu.SemaphoreType.DMA,
        ],
    )
    return fn(table, idx2d)


def _mlp_body(rows1_ref, rows2_ref, centers_ref, *refs, ctab, blk):
    out_ref = refs[-1]
    w_refs = refs[:-1]
    r1 = rows1_ref[0][:, :ctab]
    r2 = rows2_ref[0][:, :ctab]
    cpad = jnp.concatenate(
        [centers_ref[0], jnp.zeros((blk, ctab - 3), jnp.float32)], axis=1)
    g = jnp.concatenate([r1 - cpad, r2 - cpad], axis=0)
    for i in range(0, len(w_refs), 2):
        w = w_refs[i][...]
        bias = w_refs[i + 1][...]
        g = jnp.maximum(jnp.dot(g, w, preferred_element_type=jnp.float32) + bias, 0.0)
    out_ref[0] = jnp.maximum(g[:blk], g[blk:])


def _mlp_call(rows1, rows2, centers, layers, blk, ctab):
    b, npoint, dpad = rows1.shape
    cout = layers[-1][0].shape[1]
    w_args = []
    w_specs = []
    for (w, bias) in layers:
        w_args.append(w)
        w_specs.append(pl.BlockSpec(w.shape, lambda bb, j: (0, 0)))
        bias2 = bias.reshape(1, -1)
        w_args.append(bias2)
        w_specs.append(pl.BlockSpec(bias2.shape, lambda bb, j: (0, 0)))
    fn = pl.pallas_call(
        functools.partial(_mlp_body, ctab=ctab, blk=blk),
        grid=(b, npoint // blk),
        in_specs=[
            pl.BlockSpec((1, blk, dpad), lambda bb, j: (bb, j, 0)),
            pl.BlockSpec((1, blk, dpad), lambda bb, j: (bb, j, 0)),
            pl.BlockSpec((1, blk, 3), lambda bb, j: (bb, j, 0)),
            *w_specs,
        ],
        out_specs=pl.BlockSpec((1, blk, cout), lambda bb, j: (bb, j, 0)),
        out_shape=jax.ShapeDtypeStruct((b, npoint, cout), jnp.float32),
    )
    return fn(rows1, rows2, centers, *w_args)


def _set_conv(centers, post, table, layers, radius, blk):
    b, npoint, _ = centers.shape
    n = post.shape[2]
    ctab = table.shape[2]
    dpad = 128
    idx = _bq_call(centers, post, radius, blk)
    table_pad = jnp.concatenate(
        [table, jnp.zeros((b, n, dpad - ctab), jnp.float32)], axis=-1)
    rows = _sc_gather(table_pad.reshape(b * n, dpad), idx.reshape(-1), dpad)
    rows = rows.reshape(b, npoint, 2, dpad)
    return _mlp_call(rows[:, :, 0], rows[:, :, 1], centers, layers, blk, ctab)


def kernel(x, features, params):
    pos = x[:, :, :3]
    feat = jnp.transpose(features, (0, 2, 1))

    px, py, pz = pos[:, :, 0], pos[:, :, 1], pos[:, :, 2]
    out1 = _fps_call(px, py, pz, npoint=2048)
    pos2 = jnp.transpose(out1, (1, 0, 2))
    post1 = jnp.transpose(pos, (0, 2, 1))
    table1 = jnp.concatenate([pos, features], axis=-1)
    nf1 = _set_conv(pos2, post1, table1, params["l1"], radius=0.5, blk=128)
    feat2 = jnp.transpose(nf1, (0, 2, 1))

    out2 = _fps_call(pos2[:, :, 0], pos2[:, :, 1], pos2[:, :, 2],
                     npoint=512)
    pos3 = jnp.transpose(out2, (1, 0, 2))
    post2 = jnp.transpose(pos2, (0, 2, 1))
    table2 = jnp.concatenate([pos2, nf1], axis=-1)
    nf2 = _set_conv(pos3, post2, table2, params["l2"], radius=1.0, blk=128)
    feat3 = jnp.transpose(nf2, (0, 2, 1))

    return (pos, feat, pos2, feat2, pos3, feat3)
